# baseline (speedup 1.0000x reference)
import functools

import jax
import jax.numpy as jnp
from jax import lax
from jax.experimental import pallas as pl
from jax.experimental.pallas import tpu as pltpu
from jax.experimental.pallas import tpu_sc as plsc


def _prep_kernel(x_ref, wk_ref, wq_ref, w0_ref):
    xb = x_ref[...]
    wk = wk_ref[...]
    wq = wq_ref[...]
    cols = []
    for b in range(xb.shape[0]):
        kb = jax.lax.dot_general(xb[b], wk, (((1,), (1,)), ((), ())),
                                 preferred_element_type=jnp.float32)
        qb = jax.lax.dot_general(xb[b], wq, (((1,), (1,)), ((), ())),
                                 preferred_element_type=jnp.float32)
        cols.append(jnp.mean(kb * qb, axis=-1, keepdims=True))
    w0_ref[...] = jnp.concatenate(cols, axis=1)


def _matvec_kernel(c_ref, w_ref, y_ref, *, decay):
    y_ref[...] = jnp.dot(c_ref[...], w_ref[...] * decay,
                         preferred_element_type=jnp.float32)


def _sc_copy_body(src_hbm, dst_hbm, buf, *, n_chunks, cr, n_workers):
    cid = lax.axis_index("c")
    sid = lax.axis_index("s")
    wid = sid * 2 + cid

    def chunk_body(j, carry):
        chunk = wid + j * n_workers
        rows = chunk * cr
        pltpu.sync_copy(src_hbm.at[pl.ds(rows, cr)], buf)
        pltpu.sync_copy(buf, dst_hbm.at[pl.ds(rows, cr)])
        return carry

    my_n = (n_chunks - wid + n_workers - 1) // n_workers
    lax.fori_loop(0, my_n, chunk_body, 0)


def _sc_copy(connection_matrix, n, *, cr=4):
    n_chunks = n // cr
    mesh = plsc.VectorSubcoreMesh(core_axis_name="c", subcore_axis_name="s")
    body = functools.partial(_sc_copy_body, n_chunks=n_chunks, cr=cr,
                             n_workers=32)
    return pl.kernel(
        body,
        mesh=mesh,
        out_type=jax.ShapeDtypeStruct((n, n), jnp.float32),
        scratch_types=[pltpu.VMEM((cr, n), jnp.float32)],
    )(connection_matrix)


def _matvec_copy_kernel(c_ref, w_ref, y_ref, cc_ref, *, decay):
    cb = c_ref[...]
    y_ref[...] = jnp.dot(cb, w_ref[...] * decay,
                         preferred_element_type=jnp.float32)
    cc_ref[...] = cb


def _mv2_final_kernel(c_ref, w0_ref, y1_ref, x_ref, wv_ref, g_ref, bb_ref,
                      out_ref, *, decay, eps):
    i = pl.program_id(0)
    bm = c_ref.shape[0]
    y1 = y1_ref[...]
    y2 = jnp.dot(c_ref[...], y1 * decay,
                 preferred_element_type=jnp.float32)
    wtot = w0_ref[...] + y1_ref[pl.ds(i * bm, bm), :] + y2
    xb = x_ref[...]
    wv = wv_ref[...]
    g = g_ref[...]
    beta = bb_ref[...]
    for b in range(xb.shape[0]):
        vb = jax.lax.dot_general(xb[b], wv, (((1,), (1,)), ((), ())),
                                 preferred_element_type=jnp.float32)
        ob = vb * wtot[:, b:b + 1]
        mu = jnp.mean(ob, axis=-1, keepdims=True)
        var = jnp.mean((ob - mu) ** 2, axis=-1, keepdims=True)
        out_ref[b] = (ob - mu) / jnp.sqrt(var + eps) * g + beta


def kernel(x, connection_matrix, Wk, Wq, Wv, gamma, beta):
    B, N, Cin = x.shape
    KD = Wk.shape[0]
    Cout = Wv.shape[0]
    decay = 0.7
    eps = 1e-5

    BM1 = 1000
    w0 = pl.pallas_call(
        _prep_kernel,
        grid=(N // BM1,),
        in_specs=[
            pl.BlockSpec((B, BM1, Cin), lambda i: (0, i, 0)),
            pl.BlockSpec((KD, Cin), lambda i: (0, 0)),
            pl.BlockSpec((KD, Cin), lambda i: (0, 0)),
        ],
        out_specs=pl.BlockSpec((BM1, B), lambda i: (i, 0)),
        out_shape=jax.ShapeDtypeStruct((N, B), jnp.float32),
    )(x, Wk, Wq)

    BM2 = 200
    mv_copy = pl.pallas_call(
        functools.partial(_matvec_copy_kernel, decay=decay),
        grid=(N // BM2,),
        in_specs=[
            pl.BlockSpec((BM2, N), lambda i: (i, 0)),
            pl.BlockSpec((N, B), lambda i: (0, 0)),
        ],
        out_specs=[
            pl.BlockSpec((BM2, B), lambda i: (i, 0)),
            pl.BlockSpec((BM2, N), lambda i: (i, 0)),
        ],
        out_shape=[
            jax.ShapeDtypeStruct((N, B), jnp.float32),
            jax.ShapeDtypeStruct((N, N), jnp.float32),
        ],
    )
    c_copy = _sc_copy(connection_matrix, N)
    y1 = pl.pallas_call(
        functools.partial(_matvec_kernel, decay=decay),
        grid=(N // BM2,),
        in_specs=[
            pl.BlockSpec((BM2, N), lambda i: (i, 0)),
            pl.BlockSpec((N, B), lambda i: (0, 0)),
        ],
        out_specs=pl.BlockSpec((BM2, B), lambda i: (i, 0)),
        out_shape=jax.ShapeDtypeStruct((N, B), jnp.float32),
    )(connection_matrix, w0)

    out = pl.pallas_call(
        functools.partial(_mv2_final_kernel, decay=decay, eps=eps),
        grid=(N // BM2,),
        in_specs=[
            pl.BlockSpec((BM2, N), lambda i: (i, 0)),
            pl.BlockSpec((BM2, B), lambda i: (i, 0)),
            pl.BlockSpec((N, B), lambda i: (0, 0)),
            pl.BlockSpec((B, BM2, Cin), lambda i: (0, i, 0)),
            pl.BlockSpec((Cout, Cin), lambda i: (0, 0)),
            pl.BlockSpec((1, Cout), lambda i: (0, 0)),
            pl.BlockSpec((1, Cout), lambda i: (0, 0)),
        ],
        out_specs=pl.BlockSpec((B, BM2, Cout), lambda i: (0, i, 0)),
        out_shape=jax.ShapeDtypeStruct((B, N, Cout), jnp.float32),
    )(connection_matrix, w0, y1, x, Wv,
      gamma.reshape(1, Cout), beta.reshape(1, Cout))

    return (out, c_copy)


# speedup vs baseline: 1.3013x; 1.3013x over previous
import functools

import jax
import jax.numpy as jnp
from jax import lax
from jax.experimental import pallas as pl
from jax.experimental.pallas import tpu as pltpu


def _fused_kernel(x_ref, c_ref, wk_ref, wq_ref, wv_ref, g_ref, bb_ref,
                  out_ref, cc_ref, w0s_ref, y1s_ref, *, nb, bm, decay, eps):
    s = pl.program_id(0)

    @pl.when(s < nb)
    def _prep():
        xb = x_ref[...]
        wk = wk_ref[...]
        wq = wq_ref[...]
        cols = []
        for b in range(xb.shape[0]):
            kb = lax.dot_general(xb[b], wk, (((1,), (1,)), ((), ())),
                                 preferred_element_type=jnp.float32)
            qb = lax.dot_general(xb[b], wq, (((1,), (1,)), ((), ())),
                                 preferred_element_type=jnp.float32)
            cols.append(jnp.mean(kb * qb, axis=-1, keepdims=True))
        w0s_ref[pl.ds(s * bm, bm), :] = jnp.concatenate(cols, axis=1) * decay

    @pl.when((s >= nb) & (s < 2 * nb))
    def _mv1():
        i = s - nb
        cb = c_ref[...]
        y1 = jnp.dot(cb, w0s_ref[...],
                     preferred_element_type=jnp.float32)
        y1s_ref[pl.ds(i * bm, bm), :] = y1 * decay
        cc_ref[...] = cb

    @pl.when(s >= 2 * nb)
    def _mv2_final():
        i = s - 2 * nb
        cb = c_ref[...]
        y2 = jnp.dot(cb, y1s_ref[...],
                     preferred_element_type=jnp.float32)
        rows = pl.ds(i * bm, bm)
        wtot = (w0s_ref[rows, :] + y1s_ref[rows, :]) * (1.0 / decay) + y2
        xb = x_ref[...]
        wv = wv_ref[...]
        g = g_ref[...]
        beta = bb_ref[...]
        for b in range(xb.shape[0]):
            vb = lax.dot_general(xb[b], wv, (((1,), (1,)), ((), ())),
                                 preferred_element_type=jnp.float32)
            ob = vb * wtot[:, b:b + 1]
            mu = jnp.mean(ob, axis=-1, keepdims=True)
            var = jnp.mean((ob - mu) ** 2, axis=-1, keepdims=True)
            out_ref[b] = (ob - mu) / jnp.sqrt(var + eps) * g + beta


def kernel(x, connection_matrix, Wk, Wq, Wv, gamma, beta):
    B, N, Cin = x.shape
    KD = Wk.shape[0]
    Cout = Wv.shape[0]
    decay = 0.7
    eps = 1e-5

    BM = 200
    NB = N // BM

    def x_idx(s):
        return (0, jnp.where(s < NB, s, jnp.maximum(s - 2 * NB, 0)), 0)

    def c_idx(s):
        return (lax.rem(jnp.maximum(s - NB, 0), NB), 0)

    def out_idx(s):
        return (0, jnp.maximum(s - 2 * NB, 0), 0)

    def cc_idx(s):
        return (jnp.clip(s - NB, 0, NB - 1), 0)

    out, c_copy = pl.pallas_call(
        functools.partial(_fused_kernel, nb=NB, bm=BM, decay=decay, eps=eps),
        grid=(3 * NB,),
        in_specs=[
            pl.BlockSpec((B, BM, Cin), x_idx),
            pl.BlockSpec((BM, N), c_idx),
            pl.BlockSpec((KD, Cin), lambda s: (0, 0)),
            pl.BlockSpec((KD, Cin), lambda s: (0, 0)),
            pl.BlockSpec((Cout, Cin), lambda s: (0, 0)),
            pl.BlockSpec((1, Cout), lambda s: (0, 0)),
            pl.BlockSpec((1, Cout), lambda s: (0, 0)),
        ],
        out_specs=[
            pl.BlockSpec((B, BM, Cout), out_idx),
            pl.BlockSpec((BM, N), cc_idx),
        ],
        out_shape=[
            jax.ShapeDtypeStruct((B, N, Cout), jnp.float32),
            jax.ShapeDtypeStruct((N, N), jnp.float32),
        ],
        scratch_shapes=[
            pltpu.VMEM((N, B), jnp.float32),
            pltpu.VMEM((N, B), jnp.float32),
        ],
    )(x, connection_matrix, Wk, Wq, Wv,
      gamma.reshape(1, Cout), beta.reshape(1, Cout))

    return (out, c_copy)


# speedup vs baseline: 1.3645x; 1.0485x over previous
import functools

import jax
import jax.numpy as jnp
from jax import lax
from jax.experimental import pallas as pl
from jax.experimental.pallas import tpu as pltpu


def _prep_kernel(x_ref, wk_ref, wq_ref, w0_ref, *, decay):
    xb = x_ref[...]
    wk = wk_ref[...]
    wq = wq_ref[...]
    cols = []
    for b in range(xb.shape[0]):
        kb = lax.dot_general(xb[b], wk, (((1,), (1,)), ((), ())),
                             preferred_element_type=jnp.float32)
        qb = lax.dot_general(xb[b], wq, (((1,), (1,)), ((), ())),
                             preferred_element_type=jnp.float32)
        cols.append(jnp.mean(kb * qb, axis=-1, keepdims=True))
    w0_ref[...] = jnp.concatenate(cols, axis=1) * decay


def _diffusion_kernel(c_ref, w0s_ref, x_ref, wv_ref, g_ref, bb_ref,
                      out_ref, cc_ref, y1s_ref, *, nb, bm, decay, eps):
    s = pl.program_id(0)

    @pl.when(s < nb)
    def _mv1():
        cb = c_ref[...]
        y1 = jnp.dot(cb, w0s_ref[...],
                     preferred_element_type=jnp.float32)
        y1s_ref[pl.ds(s * bm, bm), :] = y1 * decay
        cc_ref[...] = cb

    @pl.when(s >= nb)
    def _mv2_final():
        i = s - nb
        cb = c_ref[...]
        y2 = jnp.dot(cb, y1s_ref[...],
                     preferred_element_type=jnp.float32)
        rows = pl.ds(i * bm, bm)
        wtot = (w0s_ref[rows, :] + y1s_ref[rows, :]) * (1.0 / decay) + y2
        xb = x_ref[...]
        wv = wv_ref[...]
        g = g_ref[...]
        beta = bb_ref[...]
        for b in range(xb.shape[0]):
            vb = lax.dot_general(xb[b], wv, (((1,), (1,)), ((), ())),
                                 preferred_element_type=jnp.float32)
            ob = vb * wtot[:, b:b + 1]
            mu = jnp.mean(ob, axis=-1, keepdims=True)
            var = jnp.mean((ob - mu) ** 2, axis=-1, keepdims=True)
            out_ref[b] = (ob - mu) / jnp.sqrt(var + eps) * g + beta


def kernel(x, connection_matrix, Wk, Wq, Wv, gamma, beta):
    B, N, Cin = x.shape
    KD = Wk.shape[0]
    Cout = Wv.shape[0]
    decay = 0.7
    eps = 1e-5

    BM1 = 1000
    w0s = pl.pallas_call(
        functools.partial(_prep_kernel, decay=decay),
        grid=(N // BM1,),
        in_specs=[
            pl.BlockSpec((B, BM1, Cin), lambda i: (0, i, 0)),
            pl.BlockSpec((KD, Cin), lambda i: (0, 0)),
            pl.BlockSpec((KD, Cin), lambda i: (0, 0)),
        ],
        out_specs=pl.BlockSpec((BM1, B), lambda i: (i, 0)),
        out_shape=jax.ShapeDtypeStruct((N, B), jnp.float32),
    )(x, Wk, Wq)

    BM = 200
    NB = N // BM

    out, c_copy = pl.pallas_call(
        functools.partial(_diffusion_kernel, nb=NB, bm=BM, decay=decay,
                          eps=eps),
        grid=(2 * NB,),
        in_specs=[
            pl.BlockSpec((BM, N), lambda s: (lax.rem(s, NB), 0)),
            pl.BlockSpec((N, B), lambda s: (0, 0)),
            pl.BlockSpec((B, BM, Cin),
                         lambda s: (0, jnp.maximum(s - NB, 0), 0)),
            pl.BlockSpec((Cout, Cin), lambda s: (0, 0)),
            pl.BlockSpec((1, Cout), lambda s: (0, 0)),
            pl.BlockSpec((1, Cout), lambda s: (0, 0)),
        ],
        out_specs=[
            pl.BlockSpec((B, BM, Cout),
                         lambda s: (0, jnp.maximum(s - NB, 0), 0)),
            pl.BlockSpec((BM, N), lambda s: (jnp.minimum(s, NB - 1), 0)),
        ],
        out_shape=[
            jax.ShapeDtypeStruct((B, N, Cout), jnp.float32),
            jax.ShapeDtypeStruct((N, N), jnp.float32),
        ],
        scratch_shapes=[
            pltpu.VMEM((N, B), jnp.float32),
        ],
    )(connection_matrix, w0s, x, Wv,
      gamma.reshape(1, Cout), beta.reshape(1, Cout))

    return (out, c_copy)
